# baseline (speedup 1.0000x reference)
import functools

import jax
import jax.numpy as jnp
from jax import lax
from jax.experimental import pallas as pl
from jax.experimental.pallas import tpu as pltpu
from jax.experimental.pallas import tpu_sc as plsc

N = 10000
E = 320000
HID = 128
IN_DIM = 64
NCLS = 8
L = 4

NUM_CORES = 2
NUM_SUBCORES = 16
NW = NUM_CORES * NUM_SUBCORES
CH = 128
CPW = 80
EPAD = CPW * NW * CH
TRASH = N
ZROWS = 632
NPAD = ZROWS * NUM_SUBCORES
OROWS = 624
OTAIL = N - OROWS * NUM_SUBCORES
HS = 40

FAST_C = 1
CF = 160
CS = 0
HSS = 16
FAST_TOT = NUM_SUBCORES * CF

_mesh = plsc.VectorSubcoreMesh(core_axis_name="c", subcore_axis_name="s")


@functools.partial(
    pl.kernel,
    out_type=jax.ShapeDtypeStruct((NUM_CORES, N, HID), jnp.float32),
    mesh=_mesh,
    scratch_types=[
        pltpu.VMEM((HSS, CH), jnp.int32),
        pltpu.VMEM((HSS, CH), jnp.int32),
        pltpu.VMEM((CH, HID), jnp.float32),
        pltpu.VMEM((CH, HID), jnp.float32),
        pltpu.VMEM_SHARED((NPAD, HID), jnp.float32),
        pltpu.SemaphoreType.DMA,
        pltpu.SemaphoreType.DMA,
    ],
)
def _sc_spmm(x_hbm, src_hbm, dst_hbm, z_hbm, out_hbm,
             srcv, dstv, rows0, rows1, acc, sem0, sem1):
    c = lax.axis_index("c")
    s = lax.axis_index("s")
    fast = c == FAST_C
    nst = jnp.where(fast, CF // HSS, CS // HSS)
    wbase = jnp.where(fast, s * CF, FAST_TOT + s * CS)

    def gather(idx, j, buf, sem):
        pltpu.async_copy(x_hbm.at[idx.at[j]], buf, sem)

    def gwait(idx, j, buf, sem):
        pltpu.make_async_copy(x_hbm.at[idx.at[j]], buf, sem).wait()

    pltpu.sync_copy(z_hbm, acc.at[pl.ds(s * ZROWS, ZROWS)])
    plsc.subcore_barrier()

    def stage_body(st, _):
        base = pl.multiple_of(wbase + st * HSS, 8)
        pltpu.sync_copy(src_hbm.at[pl.ds(base, HSS)], srcv)
        pltpu.sync_copy(dst_hbm.at[pl.ds(base, HSS)], dstv)
        gather(srcv, 0, rows0, sem0)
        gather(srcv, 1, rows1, sem1)

        def body(i, _):
            j0 = 2 * i
            gwait(srcv, j0, rows0, sem0)
            pltpu.sync_copy(rows0, acc.at[dstv.at[j0]], add=True)
            gather(srcv, j0 + 2, rows0, sem0)
            gwait(srcv, j0 + 1, rows1, sem1)
            pltpu.sync_copy(rows1, acc.at[dstv.at[j0 + 1]], add=True)
            gather(srcv, j0 + 3, rows1, sem1)
            return 0

        lax.fori_loop(0, (HSS - 2) // 2, body, 0)
        gwait(srcv, HSS - 2, rows0, sem0)
        pltpu.sync_copy(rows0, acc.at[dstv.at[HSS - 2]], add=True)
        gwait(srcv, HSS - 1, rows1, sem1)
        pltpu.sync_copy(rows1, acc.at[dstv.at[HSS - 1]], add=True)
        return 0

    lax.fori_loop(0, nst, stage_body, 0)
    plsc.subcore_barrier()
    pltpu.sync_copy(acc.at[pl.ds(s * OROWS, OROWS)],
                    out_hbm.at[c, pl.ds(s * OROWS, OROWS)])
    @pl.when(s == NUM_SUBCORES - 1)
    def _():
        base = OROWS * NUM_SUBCORES
        pltpu.sync_copy(acc.at[pl.ds(base, OTAIL)],
                        out_hbm.at[c, pl.ds(base, OTAIL)])


@functools.partial(
    pl.kernel,
    out_type=jax.ShapeDtypeStruct((NUM_CORES, N, HID), jnp.float32),
    mesh=_mesh,
    scratch_types=[
        pltpu.VMEM((CPW, CH), jnp.int32),
        pltpu.VMEM((CH, HID), jnp.float32),
        pltpu.VMEM_SHARED((NPAD, HID), jnp.float32),
    ],
)
def _sc_degree(dst_hbm, zdeg_hbm, ones_hbm, out_hbm, dstv, ones_v, acc):
    c = lax.axis_index("c")
    s = lax.axis_index("s")
    w = c * NUM_SUBCORES + s
    pltpu.sync_copy(zdeg_hbm, acc.at[pl.ds(s * ZROWS, ZROWS)])
    pltpu.sync_copy(dst_hbm.at[pl.ds(w * CPW, CPW)], dstv)
    pltpu.sync_copy(ones_hbm, ones_v)
    plsc.subcore_barrier()

    def body(j, _):
        pltpu.sync_copy(ones_v, acc.at[dstv.at[j]], add=True)
        return 0

    lax.fori_loop(0, CPW, body, 0)
    plsc.subcore_barrier()
    pltpu.sync_copy(acc.at[pl.ds(s * OROWS, OROWS)],
                    out_hbm.at[c, pl.ds(s * OROWS, OROWS)])
    @pl.when(s == NUM_SUBCORES - 1)
    def _():
        base = OROWS * NUM_SUBCORES
        pltpu.sync_copy(acc.at[pl.ds(base, OTAIL)],
                        out_hbm.at[c, pl.ds(base, OTAIL)])


def _tc_prep_body(h_ref, emb_ref, degp_ref, x_ref, rdeg_ref):
    hv = h_ref[...]
    iot = lax.broadcasted_iota(jnp.int32, (N, IN_DIM), 1)
    oh = (hv == iot).astype(jnp.float32)
    x_ref[...] = jnp.dot(oh, emb_ref[...],
                         preferred_element_type=jnp.float32,
                         precision=lax.Precision.HIGHEST)
    dp = degp_ref[...]
    d = dp[0, :, 0:1] + dp[1, :, 0:1]
    rdeg_ref[...] = 1.0 / jnp.maximum(d, 1.0)


def _tc_prep(h2, emb_h, degp):
    return pl.pallas_call(
        _tc_prep_body,
        out_shape=(
            jax.ShapeDtypeStruct((N, HID), jnp.float32),
            jax.ShapeDtypeStruct((N, 1), jnp.float32),
        ),
    )(h2, emb_h, degp)


ROWS_BLK = 2000


def _tc_update_body(p_ref, rdeg_ref, x_ref, wg_ref, wa_ref, o_ref):
    pb = p_ref[...]
    agg = (pb[0] + pb[1]) * rdeg_ref[...]
    t = jnp.dot(agg, wg_ref[...], preferred_element_type=jnp.float32,
                precision=lax.Precision.HIGHEST)
    z = 1.0 / (1.0 + jnp.exp(-t))
    o_ref[...] = x_ref[...] + z * jnp.dot(
        agg, wa_ref[...], preferred_element_type=jnp.float32,
        precision=lax.Precision.HIGHEST)


def _tc_update(p, rdeg, x, wg, wa):
    nb = N // ROWS_BLK
    return pl.pallas_call(
        _tc_update_body,
        grid=(nb,),
        in_specs=[
            pl.BlockSpec((NUM_CORES, ROWS_BLK, HID), lambda i: (0, i, 0)),
            pl.BlockSpec((ROWS_BLK, 1), lambda i: (i, 0)),
            pl.BlockSpec((ROWS_BLK, HID), lambda i: (i, 0)),
            pl.BlockSpec((HID, HID), lambda i: (0, 0)),
            pl.BlockSpec((HID, HID), lambda i: (0, 0)),
        ],
        out_specs=pl.BlockSpec((ROWS_BLK, HID), lambda i: (i, 0)),
        out_shape=jax.ShapeDtypeStruct((N, HID), jnp.float32),
    )(p, rdeg, x, wg, wa)


def _tc_mlp_body(x_ref, w1_ref, b1_ref, w2_ref, b2_ref, o_ref):
    mid = jnp.maximum(
        jnp.dot(x_ref[...], w1_ref[...],
                preferred_element_type=jnp.float32,
                precision=lax.Precision.HIGHEST) + b1_ref[...], 0.0)
    o_ref[...] = jnp.dot(mid, w2_ref[...],
                         preferred_element_type=jnp.float32,
                         precision=lax.Precision.HIGHEST) + b2_ref[...]


def _tc_mlp(x, w1, b1, w2, b2):
    nb = N // ROWS_BLK
    return pl.pallas_call(
        _tc_mlp_body,
        grid=(nb,),
        in_specs=[
            pl.BlockSpec((ROWS_BLK, HID), lambda i: (i, 0)),
            pl.BlockSpec((HID, HID // 2), lambda i: (0, 0)),
            pl.BlockSpec((1, HID // 2), lambda i: (0, 0)),
            pl.BlockSpec((HID // 2, NCLS), lambda i: (0, 0)),
            pl.BlockSpec((1, NCLS), lambda i: (0, 0)),
        ],
        out_specs=pl.BlockSpec((ROWS_BLK, NCLS), lambda i: (i, 0)),
        out_shape=jax.ShapeDtypeStruct((N, NCLS), jnp.float32),
    )(x, w1, b1, w2, b2)


def kernel(h, edge_index, e, emb_h, Wa, Wg, W1, b1, W2, b2):
    del e
    src = edge_index[0].astype(jnp.int32)
    dst = edge_index[1].astype(jnp.int32)
    pad = EPAD - E
    src_p = jnp.concatenate(
        [src, jnp.zeros((pad,), jnp.int32)]).reshape(CPW * NW, CH)
    dst_p = jnp.concatenate(
        [dst, jnp.full((pad,), TRASH, jnp.int32)]).reshape(CPW * NW, CH)
    zrow = jnp.zeros((ZROWS, HID), jnp.float32)
    ones_rows = jnp.ones((CH, HID), jnp.float32)

    hi = h.astype(jnp.int32)
    degp = _sc_degree(dst_p, zrow, ones_rows)
    x, rdeg = _tc_prep(hi.reshape(N, 1), emb_h, degp)
    for l in range(L):
        p = _sc_spmm(x, src_p, dst_p, zrow)
        x = _tc_update(p, rdeg, x, Wg[l], Wa[l])
    return _tc_mlp(x, W1, b1.reshape(1, HID // 2), W2, b2.reshape(1, NCLS))


# speedup vs baseline: 1.4356x; 1.4356x over previous
import functools

import jax
import jax.numpy as jnp
from jax import lax
from jax.experimental import pallas as pl
from jax.experimental.pallas import tpu as pltpu
from jax.experimental.pallas import tpu_sc as plsc

N = 10000
E = 320000
HID = 128
IN_DIM = 64
NCLS = 8
L = 4

NUM_CORES = 2
NUM_SUBCORES = 16
NW = NUM_CORES * NUM_SUBCORES
CH = 128
CPW = 80
EPAD = CPW * NW * CH
TRASH = N
ZROWS = 632
NPAD = ZROWS * NUM_SUBCORES
OROWS = 624
OTAIL = N - OROWS * NUM_SUBCORES
HS = 40

FAST_C = 1
CF = 152
CS = 8
HSS = 8
FAST_TOT = NUM_SUBCORES * CF

_mesh = plsc.VectorSubcoreMesh(core_axis_name="c", subcore_axis_name="s")


@functools.partial(
    pl.kernel,
    out_type=jax.ShapeDtypeStruct((NUM_CORES, N, HID), jnp.float32),
    mesh=_mesh,
    scratch_types=[
        pltpu.VMEM((HSS, CH), jnp.int32),
        pltpu.VMEM((HSS, CH), jnp.int32),
        pltpu.VMEM((CH, HID), jnp.float32),
        pltpu.VMEM((CH, HID), jnp.float32),
        pltpu.VMEM_SHARED((NPAD, HID), jnp.float32),
        pltpu.SemaphoreType.DMA,
        pltpu.SemaphoreType.DMA,
    ],
)
def _sc_spmm(x_hbm, src_hbm, dst_hbm, z_hbm, out_hbm,
             srcv, dstv, rows0, rows1, acc, sem0, sem1):
    c = lax.axis_index("c")
    s = lax.axis_index("s")
    fast = c == FAST_C
    nst = jnp.where(fast, CF // HSS, CS // HSS)
    wbase = jnp.where(fast, s * CF, FAST_TOT + s * CS)

    def gather(idx, j, buf, sem):
        pltpu.async_copy(x_hbm.at[idx.at[j]], buf, sem)

    def gwait(idx, j, buf, sem):
        pltpu.make_async_copy(x_hbm.at[idx.at[j]], buf, sem).wait()

    pltpu.sync_copy(z_hbm, acc.at[pl.ds(s * ZROWS, ZROWS)])
    plsc.subcore_barrier()

    def stage_body(st, _):
        base = pl.multiple_of(wbase + st * HSS, 8)
        pltpu.sync_copy(src_hbm.at[pl.ds(base, HSS)], srcv)
        pltpu.sync_copy(dst_hbm.at[pl.ds(base, HSS)], dstv)
        gather(srcv, 0, rows0, sem0)
        gather(srcv, 1, rows1, sem1)

        def body(i, _):
            j0 = 2 * i
            gwait(srcv, j0, rows0, sem0)
            pltpu.sync_copy(rows0, acc.at[dstv.at[j0]], add=True)
            gather(srcv, j0 + 2, rows0, sem0)
            gwait(srcv, j0 + 1, rows1, sem1)
            pltpu.sync_copy(rows1, acc.at[dstv.at[j0 + 1]], add=True)
            gather(srcv, j0 + 3, rows1, sem1)
            return 0

        lax.fori_loop(0, (HSS - 2) // 2, body, 0)
        gwait(srcv, HSS - 2, rows0, sem0)
        pltpu.sync_copy(rows0, acc.at[dstv.at[HSS - 2]], add=True)
        gwait(srcv, HSS - 1, rows1, sem1)
        pltpu.sync_copy(rows1, acc.at[dstv.at[HSS - 1]], add=True)
        return 0

    lax.fori_loop(0, nst, stage_body, 0)
    plsc.subcore_barrier()
    pltpu.sync_copy(acc.at[pl.ds(s * OROWS, OROWS)],
                    out_hbm.at[c, pl.ds(s * OROWS, OROWS)])
    @pl.when(s == NUM_SUBCORES - 1)
    def _():
        base = OROWS * NUM_SUBCORES
        pltpu.sync_copy(acc.at[pl.ds(base, OTAIL)],
                        out_hbm.at[c, pl.ds(base, OTAIL)])


@functools.partial(
    pl.kernel,
    out_type=jax.ShapeDtypeStruct((NUM_CORES, N, HID), jnp.float32),
    mesh=_mesh,
    scratch_types=[
        pltpu.VMEM((CPW, CH), jnp.int32),
        pltpu.VMEM((CH, HID), jnp.float32),
        pltpu.VMEM_SHARED((NPAD, HID), jnp.float32),
    ],
)
def _sc_degree(dst_hbm, zdeg_hbm, ones_hbm, out_hbm, dstv, ones_v, acc):
    c = lax.axis_index("c")
    s = lax.axis_index("s")
    w = c * NUM_SUBCORES + s
    pltpu.sync_copy(zdeg_hbm, acc.at[pl.ds(s * ZROWS, ZROWS)])
    pltpu.sync_copy(dst_hbm.at[pl.ds(w * CPW, CPW)], dstv)
    pltpu.sync_copy(ones_hbm, ones_v)
    plsc.subcore_barrier()

    def body(j, _):
        pltpu.sync_copy(ones_v, acc.at[dstv.at[j]], add=True)
        return 0

    lax.fori_loop(0, CPW, body, 0)
    plsc.subcore_barrier()
    pltpu.sync_copy(acc.at[pl.ds(s * OROWS, OROWS)],
                    out_hbm.at[c, pl.ds(s * OROWS, OROWS)])
    @pl.when(s == NUM_SUBCORES - 1)
    def _():
        base = OROWS * NUM_SUBCORES
        pltpu.sync_copy(acc.at[pl.ds(base, OTAIL)],
                        out_hbm.at[c, pl.ds(base, OTAIL)])


def _tc_prep_body(h_ref, emb_ref, degp_ref, x_ref, rdeg_ref):
    hv = h_ref[...]
    iot = lax.broadcasted_iota(jnp.int32, (N, IN_DIM), 1)
    oh = (hv == iot).astype(jnp.float32)
    x_ref[...] = jnp.dot(oh, emb_ref[...],
                         preferred_element_type=jnp.float32,
                         precision=lax.Precision.HIGHEST)
    dp = degp_ref[...]
    d = dp[0, :, 0:1] + dp[1, :, 0:1]
    rdeg_ref[...] = 1.0 / jnp.maximum(d, 1.0)


def _tc_prep(h2, emb_h, degp):
    return pl.pallas_call(
        _tc_prep_body,
        out_shape=(
            jax.ShapeDtypeStruct((N, HID), jnp.float32),
            jax.ShapeDtypeStruct((N, 1), jnp.float32),
        ),
    )(h2, emb_h, degp)


ROWS_BLK = 2000


def _tc_update_body(p_ref, rdeg_ref, x_ref, wg_ref, wa_ref, o_ref):
    pb = p_ref[...]
    agg = (pb[0] + pb[1]) * rdeg_ref[...]
    t = jnp.dot(agg, wg_ref[...], preferred_element_type=jnp.float32,
                precision=lax.Precision.HIGHEST)
    z = 1.0 / (1.0 + jnp.exp(-t))
    o_ref[...] = x_ref[...] + z * jnp.dot(
        agg, wa_ref[...], preferred_element_type=jnp.float32,
        precision=lax.Precision.HIGHEST)


def _tc_update(p, rdeg, x, wg, wa):
    nb = N // ROWS_BLK
    return pl.pallas_call(
        _tc_update_body,
        grid=(nb,),
        in_specs=[
            pl.BlockSpec((NUM_CORES, ROWS_BLK, HID), lambda i: (0, i, 0)),
            pl.BlockSpec((ROWS_BLK, 1), lambda i: (i, 0)),
            pl.BlockSpec((ROWS_BLK, HID), lambda i: (i, 0)),
            pl.BlockSpec((HID, HID), lambda i: (0, 0)),
            pl.BlockSpec((HID, HID), lambda i: (0, 0)),
        ],
        out_specs=pl.BlockSpec((ROWS_BLK, HID), lambda i: (i, 0)),
        out_shape=jax.ShapeDtypeStruct((N, HID), jnp.float32),
    )(p, rdeg, x, wg, wa)


def _tc_mlp_body(x_ref, w1_ref, b1_ref, w2_ref, b2_ref, o_ref):
    mid = jnp.maximum(
        jnp.dot(x_ref[...], w1_ref[...],
                preferred_element_type=jnp.float32,
                precision=lax.Precision.HIGHEST) + b1_ref[...], 0.0)
    o_ref[...] = jnp.dot(mid, w2_ref[...],
                         preferred_element_type=jnp.float32,
                         precision=lax.Precision.HIGHEST) + b2_ref[...]


def _tc_mlp(x, w1, b1, w2, b2):
    nb = N // ROWS_BLK
    return pl.pallas_call(
        _tc_mlp_body,
        grid=(nb,),
        in_specs=[
            pl.BlockSpec((ROWS_BLK, HID), lambda i: (i, 0)),
            pl.BlockSpec((HID, HID // 2), lambda i: (0, 0)),
            pl.BlockSpec((1, HID // 2), lambda i: (0, 0)),
            pl.BlockSpec((HID // 2, NCLS), lambda i: (0, 0)),
            pl.BlockSpec((1, NCLS), lambda i: (0, 0)),
        ],
        out_specs=pl.BlockSpec((ROWS_BLK, NCLS), lambda i: (i, 0)),
        out_shape=jax.ShapeDtypeStruct((N, NCLS), jnp.float32),
    )(x, w1, b1, w2, b2)


def kernel(h, edge_index, e, emb_h, Wa, Wg, W1, b1, W2, b2):
    del e
    src = edge_index[0].astype(jnp.int32)
    dst = edge_index[1].astype(jnp.int32)
    pad = EPAD - E
    src_p = jnp.concatenate(
        [src, jnp.zeros((pad,), jnp.int32)]).reshape(CPW * NW, CH)
    dst_p = jnp.concatenate(
        [dst, jnp.full((pad,), TRASH, jnp.int32)]).reshape(CPW * NW, CH)
    zrow = jnp.zeros((ZROWS, HID), jnp.float32)
    ones_rows = jnp.ones((CH, HID), jnp.float32)

    hi = h.astype(jnp.int32)
    degp = _sc_degree(dst_p, zrow, ones_rows)
    x, rdeg = _tc_prep(hi.reshape(N, 1), emb_h, degp)
    for l in range(L):
        p = _sc_spmm(x, src_p, dst_p, zrow)
        x = _tc_update(p, rdeg, x, Wg[l], Wa[l])
    return _tc_mlp(x, W1, b1.reshape(1, HID // 2), W2, b2.reshape(1, NCLS))
